# baseline (speedup 1.0000x reference)
import functools

import jax
import jax.numpy as jnp
from jax import lax
from jax.experimental import pallas as pl
from jax.experimental.pallas import tpu as pltpu
from jax.experimental.pallas import tpu_sc as plsc

N = 10000
NNZ = 320000
E = 320000
M = 1000000
D = 128

_NC = 2
_NS = 16
_NW = _NC * _NS
_MESH = plsc.VectorSubcoreMesh(
    core_axis_name="c", subcore_axis_name="s", num_cores=_NC, num_subcores=_NS
)


def _wid():
    return lax.axis_index("s") * _NC + lax.axis_index("c")


def _chunked_scatter_add(table, msrc, mdst, out_rows, nchunk, mp):
    C = 8192
    LOG2C = 13
    ACC = C + 1024
    T = 128
    WS = 2048
    RING = 256
    share = mp // _NS
    nv = share // 16
    jpc = (nchunk + 1) // 2
    cw = C // _NS
    lastrows = out_rows - (nchunk - 1) * C
    cw2 = (lastrows // _NS) & ~7
    cwl = lastrows - (_NS - 1) * cw2

    @functools.partial(
        pl.kernel,
        out_type=jax.ShapeDtypeStruct((out_rows, D), jnp.float32),
        mesh=_MESH,
        compiler_params=pltpu.CompilerParams(needs_layout_passes=False),
        scratch_types=[
            pltpu.VMEM((WS,), jnp.int32),
            pltpu.VMEM((WS,), jnp.int32),
            pltpu.VMEM((RING,), jnp.int32),
            pltpu.VMEM((RING,), jnp.int32),
            pltpu.VMEM((T,), jnp.int32),
            pltpu.VMEM((T,), jnp.int32),
            pltpu.VMEM((16,), jnp.int32),
            pltpu.VMEM((T, D), jnp.float32),
            pltpu.VMEM((64, D), jnp.float32),
            pltpu.VMEM_SHARED((ACC, D), jnp.float32),
            pltpu.SemaphoreType.DMA,
        ],
    )
    def k(tab_hbm, src_hbm, dst_hbm, out_hbm,
          dwin, swin, ringD, ringS, idxD, idxS, cspill, rows_v, zbuf, acc, sem):
        c = lax.axis_index("c")
        s = lax.axis_index("s")
        wid = s * _NC + c
        lane = lax.iota(jnp.int32, 16)
        zero16 = jnp.zeros((16,), jnp.float32)

        def zrow(j, carry):
            for t in range(D // 16):
                zbuf[j, pl.ds(t * 16, 16)] = zero16
            return carry
        lax.fori_loop(0, 64, zrow, 0)

        def fire(rbase):
            for t in range(T // 16):
                idxS[pl.ds(t * 16, 16)] = ringS[pl.ds(rbase + t * 16, 16)]
                idxD[pl.ds(t * 16, 16)] = ringD[pl.ds(rbase + t * 16, 16)]
            pltpu.async_copy(tab_hbm.at[idxS], rows_v, sem).wait()
            pltpu.sync_copy(rows_v, acc.at[idxD], add=True)

        def chunk_body(j, carry):
            k_id = 2 * j + c
            lo = k_id * C

            def zc(t, carry2):
                pltpu.sync_copy(zbuf, acc.at[pl.ds(s * (ACC // _NS) + t * 64, 64)])
                return carry2
            lax.fori_loop(0, ACC // _NS // 64, zc, 0)
            plsc.subcore_barrier()

            def vloop(v, st):
                cnt_vec, base = st

                @pl.when((v & (WS // 16 - 1)) == 0)
                def _load_window():
                    mbase = s * share + (v >> 7) * WS
                    pltpu.sync_copy(dst_hbm.at[pl.ds(mbase, WS)], dwin)
                    pltpu.sync_copy(src_hbm.at[pl.ds(mbase, WS)], swin)

                off = (v & (WS // 16 - 1)) * 16
                vd = dwin[pl.ds(off, 16)]
                vs = swin[pl.ds(off, 16)]
                m = (vd >> LOG2C) == k_id
                mi = jnp.where(m, jnp.full((16,), 1, jnp.int32),
                               jnp.zeros((16,), jnp.int32))
                pos = (cnt_vec + plsc.cumsum(mi) - 1) & (RING - 1)
                plsc.store_scatter(ringD, [pos], vd & (C - 1), mask=m)
                plsc.store_scatter(ringS, [pos], vs, mask=m)
                cnt_vec = cnt_vec + plsc.all_reduce_population_count(m)
                pend = jnp.max(cnt_vec) - base

                @pl.when(pend >= T)
                def _fire():
                    fire(base & (RING - 1))

                base = jnp.where(pend >= T, base + T, base)
                return (cnt_vec, base)

            cnt_vec, base = lax.fori_loop(
                0, nv, vloop, (jnp.zeros((16,), jnp.int32), jnp.int32(0)))

            pend_v = cnt_vec - base
            rbase = base & (RING - 1)
            for t in range(T // 16):
                l16 = lane + t * 16
                keep = l16 < pend_v
                sl = pl.ds(rbase + t * 16, 16)
                ringS[sl] = jnp.where(keep, ringS[sl], (wid * 16 + l16) & 511)
                ringD[sl] = jnp.where(keep, ringD[sl],
                                      C + ((wid * 16 + l16) & 1023))
            fire(rbase)
            plsc.subcore_barrier()

            @pl.when(k_id != nchunk - 1)
            def _wb_full():
                pltpu.sync_copy(acc.at[pl.ds(s * cw, cw)],
                                out_hbm.at[pl.ds(lo + s * cw, cw)])

            @pl.when((k_id == nchunk - 1) & (s < _NS - 1))
            def _wb_last():
                pltpu.sync_copy(acc.at[pl.ds(s * cw2, cw2)],
                                out_hbm.at[pl.ds(lo + s * cw2, cw2)])

            @pl.when((k_id == nchunk - 1) & (s == _NS - 1))
            def _wb_lastsub():
                pltpu.sync_copy(acc.at[pl.ds((_NS - 1) * cw2, cwl)],
                                out_hbm.at[pl.ds(lo + (_NS - 1) * cw2, cwl)])

            plsc.subcore_barrier()
            return carry

        lax.fori_loop(0, jpc, chunk_body, 0)

    return k(table, msrc, mdst)


def _gather_add_sc(nmp, cols, ret1):
    W = 80
    per_w = NNZ // _NW
    nwin = per_w // W

    @functools.partial(
        pl.kernel,
        out_type=jax.ShapeDtypeStruct((NNZ, D), jnp.float32),
        mesh=_MESH,
        scratch_types=[
            pltpu.VMEM((W,), jnp.int32),
            pltpu.VMEM((W, D), jnp.float32),
            pltpu.VMEM((W, D), jnp.float32),
            pltpu.SemaphoreType.DMA,
        ],
    )
    def k(nmp_hbm, cols_hbm, r1_hbm, out_hbm, idx_v, rows_v, r1_v, sem):
        base0 = _wid() * per_w

        def win(w, carry):
            base = base0 + w * W
            pltpu.sync_copy(cols_hbm.at[pl.ds(base, W)], idx_v)
            g = pltpu.async_copy(nmp_hbm.at[idx_v], rows_v, sem)
            pltpu.sync_copy(r1_hbm.at[pl.ds(base, W)], r1_v)
            g.wait()

            def addrow(j, c2):
                for t in range(D // 16):
                    sl = pl.ds(t * 16, 16)
                    rows_v[j, sl] = rows_v[j, sl] + r1_v[j, sl]
                return c2

            lax.fori_loop(0, W, addrow, 0)
            pltpu.sync_copy(rows_v, out_hbm.at[pl.ds(base, W)])
            return carry

        lax.fori_loop(0, nwin, win, 0)

    return k(nmp, cols, ret1)


def _mlp_body(x_ref, w1_ref, b1_ref, w2_ref, b2_ref, o_ref):
    x = x_ref[...]
    h = jnp.maximum(
        jnp.dot(x, w1_ref[...], preferred_element_type=jnp.float32) + b1_ref[...],
        0.0,
    )
    o_ref[...] = jnp.maximum(
        jnp.dot(h, w2_ref[...], preferred_element_type=jnp.float32) + b2_ref[...],
        0.0,
    )


def _mlp_pallas(x, W1, b1, W2, b2, blk):
    n = x.shape[0]
    grid = n // blk
    return pl.pallas_call(
        _mlp_body,
        grid=(grid,),
        in_specs=[
            pl.BlockSpec((blk, D), lambda i: (i, 0)),
            pl.BlockSpec((D, D), lambda i: (0, 0)),
            pl.BlockSpec((D,), lambda i: (0,)),
            pl.BlockSpec((D, D), lambda i: (0, 0)),
            pl.BlockSpec((D,), lambda i: (0,)),
        ],
        out_specs=pl.BlockSpec((blk, D), lambda i: (i, 0)),
        out_shape=jax.ShapeDtypeStruct((n, D), jnp.float32),
    )(x, W1, b1, W2, b2)


def kernel(tuple_values, tuple_rows, tuple_cols, edge_index, msg_src, msg_dst,
           Wn1, bn1, Wn2, bn2, Wd1, bd1, Wd2, bd2):
    tuple_rows = tuple_rows.astype(jnp.int32)
    tuple_cols = tuple_cols.astype(jnp.int32)
    edge_src = edge_index[0].astype(jnp.int32)
    edge_dst = edge_index[1].astype(jnp.int32)
    msg_src = msg_src.astype(jnp.int32)
    msg_dst = msg_dst.astype(jnp.int32)

    MP = 1048576
    EP = 327680
    msrc_p = jnp.pad(msg_src, (0, MP - M))
    mdst_p = jnp.pad(msg_dst, (0, MP - M), constant_values=-1)
    esrc_p = jnp.pad(edge_src, (0, EP - E))
    edst_p = jnp.pad(edge_dst, (0, EP - E), constant_values=-1)

    tX = _mlp_pallas(tuple_values, Wn1, bn1, Wn2, bn2, blk=1600)
    ret1 = _chunked_scatter_add(tX, msrc_p, mdst_p, NNZ, 40, MP)

    nodex = jax.ops.segment_max(tuple_values, tuple_rows, num_segments=N)
    nodex = jnp.where(jnp.isfinite(nodex), nodex, 0.0)
    nodex = _mlp_pallas(nodex, Wd1, bd1, Wd2, bd2, blk=1000)
    nodex_mp = _chunked_scatter_add(nodex, esrc_p, edst_p, N, 2, EP)
    return _gather_add_sc(nodex_mp, tuple_cols, ret1)
